# Initial kernel scaffold; baseline (speedup 1.0000x reference)
import jax
import jax.numpy as jnp
from jax.experimental import pallas as pl


def kernel(x, edge_index, W1, b1, W2, b2, temp):
    raise NotImplementedError("write your pallas kernel here")


# speedup vs baseline: 27.5137x; 27.5137x over previous
import functools
from math import comb

import numpy as np
import jax
import jax.numpy as jnp
from jax import lax
from jax.experimental import pallas as pl
from jax.experimental.pallas import tpu as pltpu
from jax.experimental.pallas import tpu_sc as plsc

N = 10000
E = 320000
DF = 128
DO = 64
K = 10

NT = 16
RPT = 640
EPT = E // NT
CHUNK = 128
NFULL = EPT // CHUNK
TAIL = EPT - NFULL * CHUNK
DEGC = 2000
NPAD = 10240


def _coef_matrix() -> np.ndarray:
    G = np.zeros((K + 1, K + 1), np.float64)
    for k in range(K + 1):
        ck = comb(K, k) / 2.0**K
        for j in range(K + 1):
            s = 0
            for m in range(0, min(j, k) + 1):
                if j - m <= K - k:
                    s += (-1) ** m * comb(k, m) * comb(K - k, j - m)
            G[j, k] = s * ck
    return G.astype(np.float32)


_GMAT = _coef_matrix()


def _mlp_body(x_ref, w1_ref, b1_ref, w2_ref, b2_ref, o_ref):
    a = jnp.dot(x_ref[...], w1_ref[...], preferred_element_type=jnp.float32)
    a = jnp.maximum(a + b1_ref[...], 0.0)
    o_ref[...] = (
        jnp.dot(a, w2_ref[...], preferred_element_type=jnp.float32) + b2_ref[...]
    )


def _lsm_body(o_ref, y_ref):
    v = o_ref[...]
    m = jnp.max(v, axis=1, keepdims=True)
    e = jnp.exp(v - m)
    s = jnp.sum(e, axis=1, keepdims=True)
    y_ref[...] = v - m - jnp.log(s)


def _i16(v):
    return jnp.zeros((16,), jnp.int32) + v


def _rsqrt16(d):
    i = plsc.bitcast(d, jnp.int32)
    i = jnp.int32(0x5F3759DF) - lax.shift_right_arithmetic(i, 1)
    y = plsc.bitcast(i, jnp.float32)
    for _ in range(3):
        y = y * (1.5 - 0.5 * d * y * y)
    return y


def _sc_body(h_hbm, src_hbm, dst_hbm, g_hbm, out_hbm,
             A0, A1, hstage,
             hist, ebuf, sidx, didx, sidx_t, didx_t, rowbuf, rowbuf_t,
             wbuf, hbuf, obuf, zbuf, accd, tbuf, dis_own, dinv_own,
             gv, sem):
    cid = lax.axis_index("c")
    t = lax.axis_index("s")
    on = cid == 0
    r0 = t * RPT
    nblk = jnp.minimum(RPT, N - r0) // 16
    ebase = t * EPT
    Z16 = jnp.zeros((16,), jnp.float32)

    @pl.when(on)
    def _():
        pltpu.sync_copy(g_hbm, gv)

        @pl.loop(0, N // 16)
        def _(i):
            hist[pl.ds(i * 16, 16)] = Z16

        for rr in range(16):
            for ff in range(4):
                zbuf[rr, pl.ds(ff * 16, 16)] = Z16

        @pl.loop(0, EPT // DEGC)
        def _(ci):
            pltpu.sync_copy(src_hbm.at[pl.ds(ebase + ci * DEGC, DEGC)], ebuf)

            @pl.loop(0, DEGC // 16)
            def _(kk):
                idx = ebuf[pl.ds(kk * 16, 16)]
                plsc.addupdate_scatter(hist, [idx], jnp.ones((16,), jnp.float32))

        pltpu.sync_copy(hist, hstage.at[t, pl.ds(0, N)])

    plsc.subcore_barrier()

    @pl.when(on)
    def _():
        pltpu.sync_copy(hstage.at[0, pl.ds(r0, RPT)], accd)
        for tt in range(1, NT):
            pltpu.sync_copy(hstage.at[tt, pl.ds(r0, RPT)], tbuf)

            @pl.loop(0, RPT // 16)
            def _(i):
                accd[pl.ds(i * 16, 16)] = accd[pl.ds(i * 16, 16)] + tbuf[pl.ds(i * 16, 16)]

        @pl.loop(0, RPT // 16)
        def _(i):
            d = accd[pl.ds(i * 16, 16)]
            m = d > 0.0
            y = _rsqrt16(jnp.where(m, d, 1.0))
            dis = jnp.where(m, y, 0.0)
            dis_own[pl.ds(i * 16, 16)] = dis
            dinv_own[pl.ds(i * 16, 16)] = dis * dis

        @pl.loop(0, nblk)
        def _(b):
            row = r0 + b * 16
            pltpu.sync_copy(h_hbm.at[pl.ds(row, 16)], hbuf)
            for r in range(16):
                dv = plsc.load_gather(dis_own, [_i16(b * 16 + r)])
                for f in range(4):
                    hbuf[r, pl.ds(f * 16, 16)] = hbuf[r, pl.ds(f * 16, 16)] * dv
            pltpu.sync_copy(hbuf, A0.at[pl.ds(row, 16)])
            pltpu.sync_copy(zbuf, A1.at[pl.ds(row, 16)])
            pltpu.sync_copy(zbuf, out_hbm.at[pl.ds(row, 16)])

    plsc.subcore_barrier()

    bufs = (A0, A1)
    for j in range(1, K + 1):
        cur, nxt = bufs

        @pl.when(on)
        def _(cur=cur, nxt=nxt):
            @pl.loop(0, NFULL)
            def _(ci):
                off = ebase + ci * CHUNK
                pltpu.sync_copy(src_hbm.at[pl.ds(off, CHUNK)], sidx)
                pltpu.sync_copy(dst_hbm.at[pl.ds(off, CHUNK)], didx)
                pltpu.async_copy(cur.at[sidx], rowbuf, sem).wait()
                pltpu.sync_copy(rowbuf, nxt.at[didx], add=True)

            off = ebase + NFULL * CHUNK
            pltpu.sync_copy(src_hbm.at[pl.ds(off, TAIL)], sidx_t)
            pltpu.sync_copy(dst_hbm.at[pl.ds(off, TAIL)], didx_t)
            pltpu.async_copy(cur.at[sidx_t], rowbuf_t, sem).wait()
            pltpu.sync_copy(rowbuf_t, nxt.at[didx_t], add=True)

        plsc.subcore_barrier()

        @pl.when(on)
        def _(cur=cur, nxt=nxt, j=j):
            gj = gv[j, pl.ds(0, 16)]

            @pl.loop(0, nblk)
            def _(b):
                row = r0 + b * 16
                pltpu.sync_copy(nxt.at[pl.ds(row, 16)], wbuf)
                pltpu.sync_copy(out_hbm.at[pl.ds(row, 16)], obuf)
                for r in range(16):
                    lrow = b * 16 + r
                    dv = plsc.load_gather(dinv_own, [_i16(lrow)])
                    for f in range(4):
                        w = wbuf[r, pl.ds(f * 16, 16)]
                        obuf[r, pl.ds(f * 16, 16)] = (
                            obuf[r, pl.ds(f * 16, 16)] + gj * w
                        )
                        if j < K:
                            wbuf[r, pl.ds(f * 16, 16)] = w * dv
                pltpu.sync_copy(obuf, out_hbm.at[pl.ds(row, 16)])
                if j < K:
                    pltpu.sync_copy(wbuf, nxt.at[pl.ds(row, 16)])
                    pltpu.sync_copy(zbuf, cur.at[pl.ds(row, 16)])

        plsc.subcore_barrier()
        bufs = (bufs[1], bufs[0])

    @pl.when(on)
    def _():
        g0 = gv[0, pl.ds(0, 16)]

        @pl.loop(0, nblk)
        def _(b):
            row = r0 + b * 16
            pltpu.sync_copy(h_hbm.at[pl.ds(row, 16)], hbuf)
            pltpu.sync_copy(out_hbm.at[pl.ds(row, 16)], obuf)
            for r in range(16):
                lrow = b * 16 + r
                dv = plsc.load_gather(dis_own, [_i16(lrow)])
                for f in range(4):
                    obuf[r, pl.ds(f * 16, 16)] = (
                        g0 * hbuf[r, pl.ds(f * 16, 16)]
                        + dv * obuf[r, pl.ds(f * 16, 16)]
                    )
            pltpu.sync_copy(obuf, out_hbm.at[pl.ds(row, 16)])


def _make_sc_bern():
    return pl.kernel(
        _sc_body,
        out_type=jax.ShapeDtypeStruct((N, DO), jnp.float32),
        mesh=plsc.VectorSubcoreMesh(core_axis_name="c", subcore_axis_name="s"),
        compiler_params=pltpu.CompilerParams(
            use_tc_tiling_on_sc=False, needs_layout_passes=False
        ),
        scratch_types=[
        pltpu.VMEM_SHARED((N, DO), jnp.float32),
        pltpu.VMEM_SHARED((N, DO), jnp.float32),
        pltpu.VMEM_SHARED((NT, NPAD), jnp.float32),
        pltpu.VMEM((N,), jnp.float32),
        pltpu.VMEM((DEGC,), jnp.int32),
        pltpu.VMEM((CHUNK,), jnp.int32),
        pltpu.VMEM((CHUNK,), jnp.int32),
        pltpu.VMEM((TAIL,), jnp.int32),
        pltpu.VMEM((TAIL,), jnp.int32),
        pltpu.VMEM((CHUNK, DO), jnp.float32),
        pltpu.VMEM((TAIL, DO), jnp.float32),
        pltpu.VMEM((16, DO), jnp.float32),
        pltpu.VMEM((16, DO), jnp.float32),
        pltpu.VMEM((16, DO), jnp.float32),
        pltpu.VMEM((16, DO), jnp.float32),
        pltpu.VMEM((RPT,), jnp.float32),
        pltpu.VMEM((RPT,), jnp.float32),
        pltpu.VMEM((RPT,), jnp.float32),
        pltpu.VMEM((RPT,), jnp.float32),
            pltpu.VMEM((16, 16), jnp.float32),
            pltpu.SemaphoreType.DMA,
        ],
    )


def kernel(x, edge_index, W1, b1, W2, b2, temp):
    h = pl.pallas_call(
        _mlp_body,
        grid=(10,),
        in_specs=[
            pl.BlockSpec((1000, DF), lambda i: (i, 0)),
            pl.BlockSpec((DF, DO), lambda i: (0, 0)),
            pl.BlockSpec((1, DO), lambda i: (0, 0)),
            pl.BlockSpec((DO, DO), lambda i: (0, 0)),
            pl.BlockSpec((1, DO), lambda i: (0, 0)),
        ],
        out_specs=pl.BlockSpec((1000, DO), lambda i: (i, 0)),
        out_shape=jax.ShapeDtypeStruct((N, DO), jnp.float32),
    )(x, W1, b1[None, :], W2, b2[None, :])

    tr = jax.nn.relu(temp)
    g = jnp.sum(jnp.asarray(_GMAT) * tr[None, :], axis=1)
    g16 = jnp.zeros((16, 16), jnp.float32).at[: K + 1, :].set(
        jnp.broadcast_to(g[:, None], (K + 1, 16))
    )

    out_lin = _make_sc_bern()(h, edge_index[0], edge_index[1], g16)

    return pl.pallas_call(
        _lsm_body,
        grid=(10,),
        in_specs=[pl.BlockSpec((1000, DO), lambda i: (i, 0))],
        out_specs=pl.BlockSpec((1000, DO), lambda i: (i, 0)),
        out_shape=jax.ShapeDtypeStruct((N, DO), jnp.float32),
    )(out_lin)


# speedup vs baseline: 30.0119x; 1.0908x over previous
from math import comb

import numpy as np
import jax
import jax.numpy as jnp
from jax import lax
from jax.experimental import pallas as pl
from jax.experimental.pallas import tpu as pltpu
from jax.experimental.pallas import tpu_sc as plsc

N = 10000
E = 320000
DF = 128
DO = 64
K = 10

NT = 16
RPT = 640
CHUNK = 128
NCH = E // CHUNK
CPT = NCH // NT
XTRA = NCH % NT
NCHPAD = NCH + 8
NPAD = 10240


def _coef_matrix() -> np.ndarray:
    G = np.zeros((K + 1, K + 1), np.float64)
    for k in range(K + 1):
        ck = comb(K, k) / 2.0**K
        for j in range(K + 1):
            s = 0
            for m in range(0, min(j, k) + 1):
                if j - m <= K - k:
                    s += (-1) ** m * comb(k, m) * comb(K - k, j - m)
            G[j, k] = s * ck
    return G.astype(np.float32)


_GMAT = _coef_matrix()


def _mlp_body(x_ref, w1_ref, b1_ref, w2_ref, b2_ref, o_ref):
    a = jnp.dot(x_ref[...], w1_ref[...], preferred_element_type=jnp.float32)
    a = jnp.maximum(a + b1_ref[...], 0.0)
    o_ref[...] = (
        jnp.dot(a, w2_ref[...], preferred_element_type=jnp.float32) + b2_ref[...]
    )


def _lsm_body(o_ref, y_ref):
    v = o_ref[...]
    m = jnp.max(v, axis=1, keepdims=True)
    e = jnp.exp(v - m)
    s = jnp.sum(e, axis=1, keepdims=True)
    y_ref[...] = v - m - jnp.log(s)


def _i16(v):
    return jnp.zeros((16,), jnp.int32) + v


def _rsqrt16(d):
    i = plsc.bitcast(d, jnp.int32)
    i = jnp.int32(0x5F3759DF) - lax.shift_right_arithmetic(i, 1)
    y = plsc.bitcast(i, jnp.float32)
    for _ in range(3):
        y = y * (1.5 - 0.5 * d * y * y)
    return y


def _sc_body(h_hbm, src2d, dst2d, g_hbm, out_hbm,
             A0, A1, hstage,
             hist, sidx0, didx0, sidx1, didx1, rowbuf0, rowbuf1,
             wbuf, hbuf, obuf, zbuf, accd, tbuf, dis_own, dinv_own,
             gv, sem0, sem1):
    cid = lax.axis_index("c")
    t = lax.axis_index("s")
    on = cid == 0
    r0 = t * RPT
    nblk = jnp.minimum(RPT, N - r0) // 16
    cbase = t * CPT + jnp.minimum(t, XTRA)
    has_extra = t < XTRA
    Z16 = jnp.zeros((16,), jnp.float32)
    ONES16 = jnp.ones((16,), jnp.float32)

    def load_idx(s_ref, d_ref, c):
        pltpu.sync_copy(src2d.at[c, pl.ds(0, CHUNK)], s_ref)
        pltpu.sync_copy(dst2d.at[c, pl.ds(0, CHUNK)], d_ref)

    @pl.when(on)
    def _():
        pltpu.sync_copy(g_hbm, gv)

        @pl.loop(0, N // 16)
        def _(i):
            hist[pl.ds(i * 16, 16)] = Z16

        for rr in range(16):
            for ff in range(4):
                zbuf[rr, pl.ds(ff * 16, 16)] = Z16

        @pl.loop(0, CPT + has_extra.astype(jnp.int32))
        def _(ci):
            pltpu.sync_copy(src2d.at[cbase + ci, pl.ds(0, CHUNK)], sidx0)

            @pl.loop(0, CHUNK // 16)
            def _(kk):
                idx = sidx0[pl.ds(kk * 16, 16)]
                plsc.addupdate_scatter(hist, [idx], ONES16)

        pltpu.sync_copy(hist, hstage.at[t, pl.ds(0, N)])

    plsc.subcore_barrier()

    @pl.when(on)
    def _():
        pltpu.sync_copy(hstage.at[0, pl.ds(r0, RPT)], accd)
        for tt in range(1, NT):
            pltpu.sync_copy(hstage.at[tt, pl.ds(r0, RPT)], tbuf)

            @pl.loop(0, RPT // 16)
            def _(i):
                accd[pl.ds(i * 16, 16)] = accd[pl.ds(i * 16, 16)] + tbuf[pl.ds(i * 16, 16)]

        @pl.loop(0, RPT // 16)
        def _(i):
            d = accd[pl.ds(i * 16, 16)]
            m = d > 0.0
            y = _rsqrt16(jnp.where(m, d, 1.0))
            dis = jnp.where(m, y, 0.0)
            dis_own[pl.ds(i * 16, 16)] = dis
            dinv_own[pl.ds(i * 16, 16)] = dis * dis

        @pl.loop(0, nblk)
        def _(b):
            row = r0 + b * 16
            pltpu.sync_copy(h_hbm.at[pl.ds(row, 16)], hbuf)
            for r in range(16):
                dv = plsc.load_gather(dis_own, [_i16(b * 16 + r)])
                for f in range(4):
                    hbuf[r, pl.ds(f * 16, 16)] = hbuf[r, pl.ds(f * 16, 16)] * dv
            pltpu.sync_copy(hbuf, A0.at[pl.ds(row, 16)])
            pltpu.sync_copy(zbuf, A1.at[pl.ds(row, 16)])
            pltpu.sync_copy(zbuf, out_hbm.at[pl.ds(row, 16)])

    plsc.subcore_barrier()

    bufs = (A0, A1)
    for j in range(1, K + 1):
        cur, nxt = bufs

        @pl.when(on)
        def _(cur=cur, nxt=nxt):
            load_idx(sidx0, didx0, cbase)
            pltpu.async_copy(cur.at[sidx0], rowbuf0, sem0)

            @pl.loop(0, CPT // 2)
            def _(i):
                c = cbase + 2 * i
                load_idx(sidx1, didx1, c + 1)
                pltpu.make_async_copy(cur.at[sidx0], rowbuf0, sem0).wait()
                pltpu.async_copy(cur.at[sidx1], rowbuf1, sem1)
                pltpu.sync_copy(rowbuf0, nxt.at[didx0], add=True)
                load_idx(sidx0, didx0, c + 2)
                pltpu.make_async_copy(cur.at[sidx1], rowbuf1, sem1).wait()
                pltpu.async_copy(cur.at[sidx0], rowbuf0, sem0)
                pltpu.sync_copy(rowbuf1, nxt.at[didx1], add=True)

            pltpu.make_async_copy(cur.at[sidx0], rowbuf0, sem0).wait()

            @pl.when(has_extra)
            def _():
                pltpu.sync_copy(rowbuf0, nxt.at[didx0], add=True)

        plsc.subcore_barrier()

        @pl.when(on)
        def _(cur=cur, nxt=nxt, j=j):
            gj = gv[j, pl.ds(0, 16)]

            @pl.loop(0, nblk)
            def _(b):
                row = r0 + b * 16
                pltpu.sync_copy(nxt.at[pl.ds(row, 16)], wbuf)
                pltpu.sync_copy(out_hbm.at[pl.ds(row, 16)], obuf)
                for r in range(16):
                    lrow = b * 16 + r
                    dv = plsc.load_gather(dinv_own, [_i16(lrow)])
                    for f in range(4):
                        w = wbuf[r, pl.ds(f * 16, 16)]
                        obuf[r, pl.ds(f * 16, 16)] = (
                            obuf[r, pl.ds(f * 16, 16)] + gj * w
                        )
                        if j < K:
                            wbuf[r, pl.ds(f * 16, 16)] = w * dv
                pltpu.sync_copy(obuf, out_hbm.at[pl.ds(row, 16)])
                if j < K:
                    pltpu.sync_copy(wbuf, nxt.at[pl.ds(row, 16)])
                    pltpu.sync_copy(zbuf, cur.at[pl.ds(row, 16)])

        plsc.subcore_barrier()
        bufs = (bufs[1], bufs[0])

    @pl.when(on)
    def _():
        g0 = gv[0, pl.ds(0, 16)]

        @pl.loop(0, nblk)
        def _(b):
            row = r0 + b * 16
            pltpu.sync_copy(h_hbm.at[pl.ds(row, 16)], hbuf)
            pltpu.sync_copy(out_hbm.at[pl.ds(row, 16)], obuf)
            for r in range(16):
                lrow = b * 16 + r
                dv = plsc.load_gather(dis_own, [_i16(lrow)])
                for f in range(4):
                    obuf[r, pl.ds(f * 16, 16)] = (
                        g0 * hbuf[r, pl.ds(f * 16, 16)]
                        + dv * obuf[r, pl.ds(f * 16, 16)]
                    )
            pltpu.sync_copy(obuf, out_hbm.at[pl.ds(row, 16)])


def _make_sc_bern():
    return pl.kernel(
        _sc_body,
        out_type=jax.ShapeDtypeStruct((N, DO), jnp.float32),
        mesh=plsc.VectorSubcoreMesh(core_axis_name="c", subcore_axis_name="s"),
        compiler_params=pltpu.CompilerParams(
            use_tc_tiling_on_sc=False, needs_layout_passes=False
        ),
        scratch_types=[
            pltpu.VMEM_SHARED((N, DO), jnp.float32),
            pltpu.VMEM_SHARED((N, DO), jnp.float32),
            pltpu.VMEM_SHARED((NT, NPAD), jnp.float32),
            pltpu.VMEM((N,), jnp.float32),
            pltpu.VMEM((CHUNK,), jnp.int32),
            pltpu.VMEM((CHUNK,), jnp.int32),
            pltpu.VMEM((CHUNK,), jnp.int32),
            pltpu.VMEM((CHUNK,), jnp.int32),
            pltpu.VMEM((CHUNK, DO), jnp.float32),
            pltpu.VMEM((CHUNK, DO), jnp.float32),
            pltpu.VMEM((16, DO), jnp.float32),
            pltpu.VMEM((16, DO), jnp.float32),
            pltpu.VMEM((16, DO), jnp.float32),
            pltpu.VMEM((16, DO), jnp.float32),
            pltpu.VMEM((RPT,), jnp.float32),
            pltpu.VMEM((RPT,), jnp.float32),
            pltpu.VMEM((RPT,), jnp.float32),
            pltpu.VMEM((RPT,), jnp.float32),
            pltpu.VMEM((16, 16), jnp.float32),
            pltpu.SemaphoreType.DMA,
            pltpu.SemaphoreType.DMA,
        ],
    )


def kernel(x, edge_index, W1, b1, W2, b2, temp):
    h = pl.pallas_call(
        _mlp_body,
        grid=(10,),
        in_specs=[
            pl.BlockSpec((1000, DF), lambda i: (i, 0)),
            pl.BlockSpec((DF, DO), lambda i: (0, 0)),
            pl.BlockSpec((1, DO), lambda i: (0, 0)),
            pl.BlockSpec((DO, DO), lambda i: (0, 0)),
            pl.BlockSpec((1, DO), lambda i: (0, 0)),
        ],
        out_specs=pl.BlockSpec((1000, DO), lambda i: (i, 0)),
        out_shape=jax.ShapeDtypeStruct((N, DO), jnp.float32),
    )(x, W1, b1[None, :], W2, b2[None, :])

    tr = jax.nn.relu(temp)
    g = jnp.sum(jnp.asarray(_GMAT) * tr[None, :], axis=1)
    g16 = jnp.zeros((16, 16), jnp.float32).at[: K + 1, :].set(
        jnp.broadcast_to(g[:, None], (K + 1, 16))
    )

    src2d = jnp.pad(edge_index[0].reshape(NCH, CHUNK), ((0, NCHPAD - NCH), (0, 0)))
    dst2d = jnp.pad(edge_index[1].reshape(NCH, CHUNK), ((0, NCHPAD - NCH), (0, 0)))

    out_lin = _make_sc_bern()(h, src2d, dst2d, g16)

    return pl.pallas_call(
        _lsm_body,
        grid=(10,),
        in_specs=[pl.BlockSpec((1000, DO), lambda i: (i, 0))],
        out_specs=pl.BlockSpec((1000, DO), lambda i: (i, 0)),
        out_shape=jax.ShapeDtypeStruct((N, DO), jnp.float32),
    )(out_lin)


# speedup vs baseline: 38.6373x; 1.2874x over previous
from math import comb

import numpy as np
import jax
import jax.numpy as jnp
from jax import lax
from jax.experimental import pallas as pl
from jax.experimental.pallas import tpu as pltpu
from jax.experimental.pallas import tpu_sc as plsc

N = 10000
E = 320000
DF = 128
DO = 64
K = 10

NC = 2
DH = DO // NC
FG = DH // 16
NT = 16
RPT = 640
CHUNK = 128
NCH = E // CHUNK
CPT = NCH // NT
XTRA = NCH % NT
NCHPAD = NCH + 8
NPAD = 10240


def _coef_matrix() -> np.ndarray:
    G = np.zeros((K + 1, K + 1), np.float64)
    for k in range(K + 1):
        ck = comb(K, k) / 2.0**K
        for j in range(K + 1):
            s = 0
            for m in range(0, min(j, k) + 1):
                if j - m <= K - k:
                    s += (-1) ** m * comb(k, m) * comb(K - k, j - m)
            G[j, k] = s * ck
    return G.astype(np.float32)


_GMAT = _coef_matrix()


def _mlp_body(x_ref, w1_ref, b1_ref, w2_ref, b2_ref, o_ref):
    a = jnp.dot(x_ref[...], w1_ref[...], preferred_element_type=jnp.float32)
    a = jnp.maximum(a + b1_ref[...], 0.0)
    o_ref[...] = (
        jnp.dot(a, w2_ref[...], preferred_element_type=jnp.float32) + b2_ref[...]
    )


def _lsm_body(o_ref, y_ref):
    v = o_ref[...]
    m = jnp.max(v, axis=1, keepdims=True)
    e = jnp.exp(v - m)
    s = jnp.sum(e, axis=1, keepdims=True)
    y_ref[...] = v - m - jnp.log(s)


def _i16(v):
    return jnp.zeros((16,), jnp.int32) + v


def _rsqrt16(d):
    i = plsc.bitcast(d, jnp.int32)
    i = jnp.int32(0x5F3759DF) - lax.shift_right_arithmetic(i, 1)
    y = plsc.bitcast(i, jnp.float32)
    for _ in range(3):
        y = y * (1.5 - 0.5 * d * y * y)
    return y


def _sc_body(h_hbm, src2d, dst2d, g_hbm, out_hbm,
             A0, A1, hstage,
             hist, sidx0, didx0, sidx1, didx1, rowbuf0, rowbuf1,
             wbuf, hbuf, obuf, zbuf, accd, tbuf, dis_own, dinv_own,
             gv, sem0, sem1):
    cid = lax.axis_index("c")
    t = lax.axis_index("s")
    r0 = t * RPT
    nblk = jnp.minimum(RPT, N - r0) // 16
    cbase = t * CPT + jnp.minimum(t, XTRA)
    has_extra = t < XTRA
    Z16 = jnp.zeros((16,), jnp.float32)
    ONES16 = jnp.ones((16,), jnp.float32)

    def load_idx(s_ref, d_ref, c):
        pltpu.sync_copy(src2d.at[c, pl.ds(0, CHUNK)], s_ref)
        pltpu.sync_copy(dst2d.at[c, pl.ds(0, CHUNK)], d_ref)

    pltpu.sync_copy(g_hbm, gv)

    @pl.loop(0, N // 16)
    def _(i):
        hist[pl.ds(i * 16, 16)] = Z16

    for rr in range(16):
        for ff in range(FG):
            zbuf[rr, pl.ds(ff * 16, 16)] = Z16

    @pl.loop(0, CPT + has_extra.astype(jnp.int32))
    def _(ci):
        pltpu.sync_copy(src2d.at[cbase + ci, pl.ds(0, CHUNK)], sidx0)

        @pl.loop(0, CHUNK // 16)
        def _(kk):
            idx = sidx0[pl.ds(kk * 16, 16)]
            plsc.addupdate_scatter(hist, [idx], ONES16)

    pltpu.sync_copy(hist, hstage.at[t, pl.ds(0, N)])

    plsc.subcore_barrier()

    pltpu.sync_copy(hstage.at[0, pl.ds(r0, RPT)], accd)
    for tt in range(1, NT):
        pltpu.sync_copy(hstage.at[tt, pl.ds(r0, RPT)], tbuf)

        @pl.loop(0, RPT // 16)
        def _(i):
            accd[pl.ds(i * 16, 16)] = accd[pl.ds(i * 16, 16)] + tbuf[pl.ds(i * 16, 16)]

    @pl.loop(0, RPT // 16)
    def _(i):
        d = accd[pl.ds(i * 16, 16)]
        m = d > 0.0
        y = _rsqrt16(jnp.where(m, d, 1.0))
        dis = jnp.where(m, y, 0.0)
        dis_own[pl.ds(i * 16, 16)] = dis
        dinv_own[pl.ds(i * 16, 16)] = dis * dis

    @pl.loop(0, nblk)
    def _(b):
        row = r0 + b * 16
        pltpu.sync_copy(h_hbm.at[cid, pl.ds(row, 16)], hbuf)
        for r in range(16):
            dv = plsc.load_gather(dis_own, [_i16(b * 16 + r)])
            for f in range(FG):
                hbuf[r, pl.ds(f * 16, 16)] = hbuf[r, pl.ds(f * 16, 16)] * dv
        pltpu.sync_copy(hbuf, A0.at[pl.ds(row, 16)])
        pltpu.sync_copy(zbuf, A1.at[pl.ds(row, 16)])
        pltpu.sync_copy(zbuf, out_hbm.at[cid, pl.ds(row, 16)])

    plsc.subcore_barrier()

    bufs = (A0, A1)
    for j in range(1, K + 1):
        cur, nxt = bufs

        load_idx(sidx0, didx0, cbase)
        pltpu.async_copy(cur.at[sidx0], rowbuf0, sem0)

        @pl.loop(0, CPT // 2)
        def _(i, cur=cur, nxt=nxt):
            c = cbase + 2 * i
            load_idx(sidx1, didx1, c + 1)
            pltpu.make_async_copy(cur.at[sidx0], rowbuf0, sem0).wait()
            pltpu.async_copy(cur.at[sidx1], rowbuf1, sem1)
            pltpu.sync_copy(rowbuf0, nxt.at[didx0], add=True)
            load_idx(sidx0, didx0, c + 2)
            pltpu.make_async_copy(cur.at[sidx1], rowbuf1, sem1).wait()
            pltpu.async_copy(cur.at[sidx0], rowbuf0, sem0)
            pltpu.sync_copy(rowbuf1, nxt.at[didx1], add=True)

        pltpu.make_async_copy(cur.at[sidx0], rowbuf0, sem0).wait()

        @pl.when(has_extra)
        def _(cur=cur, nxt=nxt):
            pltpu.sync_copy(rowbuf0, nxt.at[didx0], add=True)

        plsc.subcore_barrier()

        gj = gv[j, pl.ds(0, 16)]

        @pl.loop(0, nblk)
        def _(b, cur=cur, nxt=nxt, j=j, gj=gj):
            row = r0 + b * 16
            pltpu.sync_copy(nxt.at[pl.ds(row, 16)], wbuf)
            pltpu.sync_copy(out_hbm.at[cid, pl.ds(row, 16)], obuf)
            for r in range(16):
                lrow = b * 16 + r
                dv = plsc.load_gather(dinv_own, [_i16(lrow)])
                for f in range(FG):
                    w = wbuf[r, pl.ds(f * 16, 16)]
                    obuf[r, pl.ds(f * 16, 16)] = (
                        obuf[r, pl.ds(f * 16, 16)] + gj * w
                    )
                    if j < K:
                        wbuf[r, pl.ds(f * 16, 16)] = w * dv
            pltpu.sync_copy(obuf, out_hbm.at[cid, pl.ds(row, 16)])
            if j < K:
                pltpu.sync_copy(wbuf, nxt.at[pl.ds(row, 16)])
                pltpu.sync_copy(zbuf, cur.at[pl.ds(row, 16)])

        plsc.subcore_barrier()
        bufs = (bufs[1], bufs[0])

    g0 = gv[0, pl.ds(0, 16)]

    @pl.loop(0, nblk)
    def _(b, g0=g0):
        row = r0 + b * 16
        pltpu.sync_copy(h_hbm.at[cid, pl.ds(row, 16)], hbuf)
        pltpu.sync_copy(out_hbm.at[cid, pl.ds(row, 16)], obuf)
        for r in range(16):
            lrow = b * 16 + r
            dv = plsc.load_gather(dis_own, [_i16(lrow)])
            for f in range(FG):
                obuf[r, pl.ds(f * 16, 16)] = (
                    g0 * hbuf[r, pl.ds(f * 16, 16)]
                    + dv * obuf[r, pl.ds(f * 16, 16)]
                )
        pltpu.sync_copy(obuf, out_hbm.at[cid, pl.ds(row, 16)])


def _make_sc_bern():
    return pl.kernel(
        _sc_body,
        out_type=jax.ShapeDtypeStruct((NC, N, DH), jnp.float32),
        mesh=plsc.VectorSubcoreMesh(core_axis_name="c", subcore_axis_name="s"),
        compiler_params=pltpu.CompilerParams(
            use_tc_tiling_on_sc=False, needs_layout_passes=False
        ),
        scratch_types=[
            pltpu.VMEM_SHARED((N, DH), jnp.float32),
            pltpu.VMEM_SHARED((N, DH), jnp.float32),
            pltpu.VMEM_SHARED((NT, NPAD), jnp.float32),
            pltpu.VMEM((N,), jnp.float32),
            pltpu.VMEM((CHUNK,), jnp.int32),
            pltpu.VMEM((CHUNK,), jnp.int32),
            pltpu.VMEM((CHUNK,), jnp.int32),
            pltpu.VMEM((CHUNK,), jnp.int32),
            pltpu.VMEM((CHUNK, DH), jnp.float32),
            pltpu.VMEM((CHUNK, DH), jnp.float32),
            pltpu.VMEM((16, DH), jnp.float32),
            pltpu.VMEM((16, DH), jnp.float32),
            pltpu.VMEM((16, DH), jnp.float32),
            pltpu.VMEM((16, DH), jnp.float32),
            pltpu.VMEM((RPT,), jnp.float32),
            pltpu.VMEM((RPT,), jnp.float32),
            pltpu.VMEM((RPT,), jnp.float32),
            pltpu.VMEM((RPT,), jnp.float32),
            pltpu.VMEM((16, 16), jnp.float32),
            pltpu.SemaphoreType.DMA,
            pltpu.SemaphoreType.DMA,
        ],
    )


def kernel(x, edge_index, W1, b1, W2, b2, temp):
    h = pl.pallas_call(
        _mlp_body,
        grid=(10,),
        in_specs=[
            pl.BlockSpec((1000, DF), lambda i: (i, 0)),
            pl.BlockSpec((DF, DO), lambda i: (0, 0)),
            pl.BlockSpec((1, DO), lambda i: (0, 0)),
            pl.BlockSpec((DO, DO), lambda i: (0, 0)),
            pl.BlockSpec((1, DO), lambda i: (0, 0)),
        ],
        out_specs=pl.BlockSpec((1000, DO), lambda i: (i, 0)),
        out_shape=jax.ShapeDtypeStruct((N, DO), jnp.float32),
    )(x, W1, b1[None, :], W2, b2[None, :])

    tr = jax.nn.relu(temp)
    g = jnp.sum(jnp.asarray(_GMAT) * tr[None, :], axis=1)
    g16 = jnp.zeros((16, 16), jnp.float32).at[: K + 1, :].set(
        jnp.broadcast_to(g[:, None], (K + 1, 16))
    )

    h2 = h.reshape(N, NC, DH).transpose(1, 0, 2)
    src2d = jnp.pad(edge_index[0].reshape(NCH, CHUNK), ((0, NCHPAD - NCH), (0, 0)))
    dst2d = jnp.pad(edge_index[1].reshape(NCH, CHUNK), ((0, NCHPAD - NCH), (0, 0)))

    out2 = _make_sc_bern()(h2, src2d, dst2d, g16)
    out_lin = out2.transpose(1, 0, 2).reshape(N, DO)

    return pl.pallas_call(
        _lsm_body,
        grid=(10,),
        in_specs=[pl.BlockSpec((1000, DO), lambda i: (i, 0))],
        out_specs=pl.BlockSpec((1000, DO), lambda i: (i, 0)),
        out_shape=jax.ShapeDtypeStruct((N, DO), jnp.float32),
    )(out_lin)


# speedup vs baseline: 43.5926x; 1.1283x over previous
from math import comb

import numpy as np
import jax
import jax.numpy as jnp
from jax import lax
from jax.experimental import pallas as pl
from jax.experimental.pallas import tpu as pltpu
from jax.experimental.pallas import tpu_sc as plsc

N = 10000
E = 320000
DF = 128
DO = 64
K = 10

NC = 2
DH = DO // NC
FG = DH // 16
NT = 16
RPT = 640
RB = 80
CHUNK = 128
NCH = E // CHUNK
CPT = NCH // NT
XTRA = NCH % NT
NCHPAD = NCH + 8
NPAD = 10240


def _coef_matrix() -> np.ndarray:
    G = np.zeros((K + 1, K + 1), np.float64)
    for k in range(K + 1):
        ck = comb(K, k) / 2.0**K
        for j in range(K + 1):
            s = 0
            for m in range(0, min(j, k) + 1):
                if j - m <= K - k:
                    s += (-1) ** m * comb(k, m) * comb(K - k, j - m)
            G[j, k] = s * ck
    return G.astype(np.float32)


_GMAT = _coef_matrix()


def _mlp_body(x_ref, w1_ref, b1_ref, w2_ref, b2_ref, o_ref):
    a = jnp.dot(x_ref[...], w1_ref[...], preferred_element_type=jnp.float32)
    a = jnp.maximum(a + b1_ref[...], 0.0)
    o_ref[...] = (
        jnp.dot(a, w2_ref[...], preferred_element_type=jnp.float32) + b2_ref[...]
    )


def _lsm_body(o_ref, y_ref):
    v = o_ref[...]
    m = jnp.max(v, axis=1, keepdims=True)
    e = jnp.exp(v - m)
    s = jnp.sum(e, axis=1, keepdims=True)
    y_ref[...] = v - m - jnp.log(s)


def _i16(v):
    return jnp.zeros((16,), jnp.int32) + v


def _rsqrt16(d):
    i = plsc.bitcast(d, jnp.int32)
    i = jnp.int32(0x5F3759DF) - lax.shift_right_arithmetic(i, 1)
    y = plsc.bitcast(i, jnp.float32)
    for _ in range(3):
        y = y * (1.5 - 0.5 * d * y * y)
    return y


def _sc_body(h_hbm, src2d, dst2d, g_hbm, out_hbm,
             A0, A1, hstage,
             hist, sidx0, didx0, sidx1, didx1, rowbuf0, rowbuf1,
             wbuf, hbuf, obuf, zbuf, accd, tbuf, dis_own, dinv_own,
             gv, sem0, sem1):
    cid = lax.axis_index("c")
    t = lax.axis_index("s")
    r0 = t * RPT
    nblk = jnp.minimum(RPT, N - r0) // RB
    cbase = t * CPT + jnp.minimum(t, XTRA)
    has_extra = t < XTRA
    Z16 = jnp.zeros((16,), jnp.float32)
    ONES16 = jnp.ones((16,), jnp.float32)

    def load_idx(s_ref, d_ref, c):
        pltpu.sync_copy(src2d.at[c, pl.ds(0, CHUNK)], s_ref)
        pltpu.sync_copy(dst2d.at[c, pl.ds(0, CHUNK)], d_ref)

    pltpu.sync_copy(g_hbm, gv)

    @pl.loop(0, N // 16)
    def _(i):
        hist[pl.ds(i * 16, 16)] = Z16

    @pl.loop(0, (RB * DH) // 16)
    def _(i):
        zbuf[i // FG, pl.ds((i % FG) * 16, 16)] = Z16

    @pl.loop(0, CPT + has_extra.astype(jnp.int32))
    def _(ci):
        pltpu.sync_copy(src2d.at[cbase + ci, pl.ds(0, CHUNK)], sidx0)

        @pl.loop(0, CHUNK // 16)
        def _(kk):
            idx = sidx0[pl.ds(kk * 16, 16)]
            plsc.addupdate_scatter(hist, [idx], ONES16)

    pltpu.sync_copy(hist, hstage.at[t, pl.ds(0, N)])

    plsc.subcore_barrier()

    pltpu.sync_copy(hstage.at[0, pl.ds(r0, RPT)], accd)
    for tt in range(1, NT):
        pltpu.sync_copy(hstage.at[tt, pl.ds(r0, RPT)], tbuf)

        @pl.loop(0, RPT // 16)
        def _(i):
            accd[pl.ds(i * 16, 16)] = accd[pl.ds(i * 16, 16)] + tbuf[pl.ds(i * 16, 16)]

    @pl.loop(0, RPT // 16)
    def _(i):
        d = accd[pl.ds(i * 16, 16)]
        m = d > 0.0
        y = _rsqrt16(jnp.where(m, d, 1.0))
        dis = jnp.where(m, y, 0.0)
        dis_own[pl.ds(i * 16, 16)] = dis
        dinv_own[pl.ds(i * 16, 16)] = dis * dis

    @pl.loop(0, nblk)
    def _(b):
        row = r0 + b * RB
        pltpu.sync_copy(h_hbm.at[cid, pl.ds(row, RB)], hbuf)

        @pl.loop(0, RB)
        def _(rr):
            dv = plsc.load_gather(dis_own, [_i16(b * RB + rr)])
            for f in range(FG):
                hbuf[rr, pl.ds(f * 16, 16)] = hbuf[rr, pl.ds(f * 16, 16)] * dv
        pltpu.sync_copy(hbuf, A0.at[pl.ds(row, RB)])
        pltpu.sync_copy(zbuf, A1.at[pl.ds(row, RB)])
        pltpu.sync_copy(zbuf, out_hbm.at[cid, pl.ds(row, RB)])

    plsc.subcore_barrier()

    bufs = (A0, A1)
    for j in range(1, K + 1):
        cur, nxt = bufs

        load_idx(sidx0, didx0, cbase)
        pltpu.async_copy(cur.at[sidx0], rowbuf0, sem0)

        @pl.loop(0, CPT // 2)
        def _(i, cur=cur, nxt=nxt):
            c = cbase + 2 * i
            load_idx(sidx1, didx1, c + 1)
            pltpu.make_async_copy(cur.at[sidx0], rowbuf0, sem0).wait()
            pltpu.async_copy(cur.at[sidx1], rowbuf1, sem1)
            pltpu.sync_copy(rowbuf0, nxt.at[didx0], add=True)
            load_idx(sidx0, didx0, c + 2)
            pltpu.make_async_copy(cur.at[sidx1], rowbuf1, sem1).wait()
            pltpu.async_copy(cur.at[sidx0], rowbuf0, sem0)
            pltpu.sync_copy(rowbuf1, nxt.at[didx1], add=True)

        pltpu.make_async_copy(cur.at[sidx0], rowbuf0, sem0).wait()

        @pl.when(has_extra)
        def _(cur=cur, nxt=nxt):
            pltpu.sync_copy(rowbuf0, nxt.at[didx0], add=True)

        plsc.subcore_barrier()

        gj = gv[j, pl.ds(0, 16)]

        @pl.loop(0, nblk)
        def _(b, cur=cur, nxt=nxt, j=j, gj=gj):
            row = r0 + b * RB
            pltpu.sync_copy(nxt.at[pl.ds(row, RB)], wbuf)
            pltpu.sync_copy(out_hbm.at[cid, pl.ds(row, RB)], obuf)

            @pl.loop(0, RB)
            def _(rr):
                dv = plsc.load_gather(dinv_own, [_i16(b * RB + rr)])
                for f in range(FG):
                    w = wbuf[rr, pl.ds(f * 16, 16)]
                    obuf[rr, pl.ds(f * 16, 16)] = obuf[rr, pl.ds(f * 16, 16)] + gj * w
                    if j < K:
                        wbuf[rr, pl.ds(f * 16, 16)] = w * dv
            pltpu.sync_copy(obuf, out_hbm.at[cid, pl.ds(row, RB)])
            if j < K:
                pltpu.sync_copy(wbuf, nxt.at[pl.ds(row, RB)])
                pltpu.sync_copy(zbuf, cur.at[pl.ds(row, RB)])

        plsc.subcore_barrier()
        bufs = (bufs[1], bufs[0])

    g0 = gv[0, pl.ds(0, 16)]

    @pl.loop(0, nblk)
    def _(b, g0=g0):
        row = r0 + b * RB
        pltpu.sync_copy(h_hbm.at[cid, pl.ds(row, RB)], hbuf)
        pltpu.sync_copy(out_hbm.at[cid, pl.ds(row, RB)], obuf)

        @pl.loop(0, RB)
        def _(rr):
            dv = plsc.load_gather(dis_own, [_i16(b * RB + rr)])
            for f in range(FG):
                obuf[rr, pl.ds(f * 16, 16)] = (
                    g0 * hbuf[rr, pl.ds(f * 16, 16)]
                    + dv * obuf[rr, pl.ds(f * 16, 16)]
                )
        pltpu.sync_copy(obuf, out_hbm.at[cid, pl.ds(row, RB)])


def _make_sc_bern():
    return pl.kernel(
        _sc_body,
        out_type=jax.ShapeDtypeStruct((NC, N, DH), jnp.float32),
        mesh=plsc.VectorSubcoreMesh(core_axis_name="c", subcore_axis_name="s"),
        compiler_params=pltpu.CompilerParams(
            use_tc_tiling_on_sc=False, needs_layout_passes=False
        ),
        scratch_types=[
            pltpu.VMEM_SHARED((N, DH), jnp.float32),
            pltpu.VMEM_SHARED((N, DH), jnp.float32),
            pltpu.VMEM_SHARED((NT, NPAD), jnp.float32),
            pltpu.VMEM((N,), jnp.float32),
            pltpu.VMEM((CHUNK,), jnp.int32),
            pltpu.VMEM((CHUNK,), jnp.int32),
            pltpu.VMEM((CHUNK,), jnp.int32),
            pltpu.VMEM((CHUNK,), jnp.int32),
            pltpu.VMEM((CHUNK, DH), jnp.float32),
            pltpu.VMEM((CHUNK, DH), jnp.float32),
            pltpu.VMEM((RB, DH), jnp.float32),
            pltpu.VMEM((RB, DH), jnp.float32),
            pltpu.VMEM((RB, DH), jnp.float32),
            pltpu.VMEM((RB, DH), jnp.float32),
            pltpu.VMEM((RPT,), jnp.float32),
            pltpu.VMEM((RPT,), jnp.float32),
            pltpu.VMEM((RPT,), jnp.float32),
            pltpu.VMEM((RPT,), jnp.float32),
            pltpu.VMEM((16, 16), jnp.float32),
            pltpu.SemaphoreType.DMA,
            pltpu.SemaphoreType.DMA,
        ],
    )


def kernel(x, edge_index, W1, b1, W2, b2, temp):
    h = pl.pallas_call(
        _mlp_body,
        grid=(10,),
        in_specs=[
            pl.BlockSpec((1000, DF), lambda i: (i, 0)),
            pl.BlockSpec((DF, DO), lambda i: (0, 0)),
            pl.BlockSpec((1, DO), lambda i: (0, 0)),
            pl.BlockSpec((DO, DO), lambda i: (0, 0)),
            pl.BlockSpec((1, DO), lambda i: (0, 0)),
        ],
        out_specs=pl.BlockSpec((1000, DO), lambda i: (i, 0)),
        out_shape=jax.ShapeDtypeStruct((N, DO), jnp.float32),
    )(x, W1, b1[None, :], W2, b2[None, :])

    tr = jax.nn.relu(temp)
    g = jnp.sum(jnp.asarray(_GMAT) * tr[None, :], axis=1)
    g16 = jnp.zeros((16, 16), jnp.float32).at[: K + 1, :].set(
        jnp.broadcast_to(g[:, None], (K + 1, 16))
    )

    h2 = h.reshape(N, NC, DH).transpose(1, 0, 2)
    src2d = jnp.pad(edge_index[0].reshape(NCH, CHUNK), ((0, NCHPAD - NCH), (0, 0)))
    dst2d = jnp.pad(edge_index[1].reshape(NCH, CHUNK), ((0, NCHPAD - NCH), (0, 0)))

    out2 = _make_sc_bern()(h2, src2d, dst2d, g16)
    out_lin = out2.transpose(1, 0, 2).reshape(N, DO)

    return pl.pallas_call(
        _lsm_body,
        grid=(10,),
        in_specs=[pl.BlockSpec((1000, DO), lambda i: (i, 0))],
        out_specs=pl.BlockSpec((1000, DO), lambda i: (i, 0)),
        out_shape=jax.ShapeDtypeStruct((N, DO), jnp.float32),
    )(out_lin)


# speedup vs baseline: 61.4304x; 1.4092x over previous
from math import comb

import numpy as np
import jax
import jax.numpy as jnp
from jax import lax
from jax.experimental import pallas as pl
from jax.experimental.pallas import tpu as pltpu
from jax.experimental.pallas import tpu_sc as plsc

N = 10000
E = 320000
DF = 128
DO = 64
K = 10

NC = 2
DH = DO // NC
FG = DH // 16
NT = 16
RPT = 640
RB = 80
CHUNK = 128
NCH = E // CHUNK
CPT = NCH // NT
XTRA = NCH % NT
NCHPAD = NCH + 8
NPAD = 10240


def _coef_matrix() -> np.ndarray:
    G = np.zeros((K + 1, K + 1), np.float64)
    for k in range(K + 1):
        ck = comb(K, k) / 2.0**K
        for j in range(K + 1):
            s = 0
            for m in range(0, min(j, k) + 1):
                if j - m <= K - k:
                    s += (-1) ** m * comb(k, m) * comb(K - k, j - m)
            G[j, k] = s * ck
    return G.astype(np.float32)


_GMAT = _coef_matrix()


def _mlp_body(x_ref, w1_ref, b1_ref, w2_ref, b2_ref, o_ref):
    a = jnp.dot(x_ref[...], w1_ref[...], preferred_element_type=jnp.float32)
    a = jnp.maximum(a + b1_ref[...], 0.0)
    o_ref[...] = (
        jnp.dot(a, w2_ref[...], preferred_element_type=jnp.float32) + b2_ref[...]
    )


def _lsm_body(o_ref, y_ref):
    v = o_ref[...]
    m = jnp.max(v, axis=1, keepdims=True)
    e = jnp.exp(v - m)
    s = jnp.sum(e, axis=1, keepdims=True)
    y_ref[...] = v - m - jnp.log(s)


def _i16(v):
    return jnp.zeros((16,), jnp.int32) + v


def _rsqrt16(d):
    i = plsc.bitcast(d, jnp.int32)
    i = jnp.int32(0x5F3759DF) - lax.shift_right_arithmetic(i, 1)
    y = plsc.bitcast(i, jnp.float32)
    for _ in range(3):
        y = y * (1.5 - 0.5 * d * y * y)
    return y


def _sc_body(h_hbm, ei_hbm, g_hbm, out_hbm,
             A0, A1, hstage,
             hist, idx0, idx1, rowbuf0, rowbuf1,
             wbuf, hbuf, obuf, zbuf, accd, tbuf, dis_own, dinv_own,
             gv, sem0, sem1):
    cid = lax.axis_index("c")
    t = lax.axis_index("s")
    r0 = t * RPT
    nblk = jnp.minimum(RPT, N - r0) // RB
    cbase = t * CPT + jnp.minimum(t, XTRA)
    has_extra = t < XTRA
    Z16 = jnp.zeros((16,), jnp.float32)
    ONES16 = jnp.ones((16,), jnp.float32)

    def load_idx(sd_ref, c):
        pltpu.sync_copy(ei_hbm.at[c], sd_ref)

    pltpu.sync_copy(g_hbm, gv)

    @pl.loop(0, N // 16)
    def _(i):
        hist[pl.ds(i * 16, 16)] = Z16

    @pl.loop(0, (RB * DH) // 16)
    def _(i):
        zbuf[i // FG, pl.ds((i % FG) * 16, 16)] = Z16

    @pl.loop(0, CPT + has_extra.astype(jnp.int32))
    def _(ci):
        pltpu.sync_copy(ei_hbm.at[cbase + ci], idx0)

        @pl.loop(0, CHUNK // 16)
        def _(kk):
            idx = idx0[0, pl.ds(kk * 16, 16)]
            plsc.addupdate_scatter(hist, [idx], ONES16)

    pltpu.sync_copy(hist, hstage.at[t, pl.ds(0, N)])

    plsc.subcore_barrier()

    pltpu.sync_copy(hstage.at[0, pl.ds(r0, RPT)], accd)
    for tt in range(1, NT):
        pltpu.sync_copy(hstage.at[tt, pl.ds(r0, RPT)], tbuf)

        @pl.loop(0, RPT // 16)
        def _(i):
            accd[pl.ds(i * 16, 16)] = accd[pl.ds(i * 16, 16)] + tbuf[pl.ds(i * 16, 16)]

    @pl.loop(0, RPT // 16)
    def _(i):
        d = accd[pl.ds(i * 16, 16)]
        m = d > 0.0
        y = _rsqrt16(jnp.where(m, d, 1.0))
        dis = jnp.where(m, y, 0.0)
        dis_own[pl.ds(i * 16, 16)] = dis
        dinv_own[pl.ds(i * 16, 16)] = dis * dis

    @pl.loop(0, nblk)
    def _(b):
        row = r0 + b * RB
        pltpu.sync_copy(h_hbm.at[cid, pl.ds(row, RB)], hbuf)

        @pl.loop(0, RB)
        def _(rr):
            dv = plsc.load_gather(dis_own, [_i16(b * RB + rr)])
            for f in range(FG):
                hbuf[rr, pl.ds(f * 16, 16)] = hbuf[rr, pl.ds(f * 16, 16)] * dv
        pltpu.sync_copy(hbuf, A0.at[pl.ds(row, RB)])
        pltpu.sync_copy(zbuf, A1.at[pl.ds(row, RB)])
        pltpu.sync_copy(zbuf, out_hbm.at[cid, pl.ds(row, RB)])

    plsc.subcore_barrier()

    bufs = (A0, A1)
    for j in range(1, K + 1):
        cur, nxt = bufs

        load_idx(idx0, cbase)
        pltpu.async_copy(cur.at[idx0.at[0]], rowbuf0, sem0)

        @pl.loop(0, CPT // 2)
        def _(i, cur=cur, nxt=nxt):
            c = cbase + 2 * i
            load_idx(idx1, c + 1)
            pltpu.make_async_copy(cur.at[idx0.at[0]], rowbuf0, sem0).wait()
            pltpu.async_copy(cur.at[idx1.at[0]], rowbuf1, sem1)
            pltpu.sync_copy(rowbuf0, nxt.at[idx0.at[1]], add=True)
            load_idx(idx0, c + 2)
            pltpu.make_async_copy(cur.at[idx1.at[0]], rowbuf1, sem1).wait()
            pltpu.async_copy(cur.at[idx0.at[0]], rowbuf0, sem0)
            pltpu.sync_copy(rowbuf1, nxt.at[idx1.at[1]], add=True)

        pltpu.make_async_copy(cur.at[idx0.at[0]], rowbuf0, sem0).wait()

        @pl.when(has_extra)
        def _(cur=cur, nxt=nxt):
            pltpu.sync_copy(rowbuf0, nxt.at[idx0.at[1]], add=True)

        plsc.subcore_barrier()

        gj = gv[j, pl.ds(0, 16)]

        @pl.loop(0, nblk)
        def _(b, cur=cur, nxt=nxt, j=j, gj=gj):
            row = r0 + b * RB
            pltpu.sync_copy(nxt.at[pl.ds(row, RB)], wbuf)
            pltpu.sync_copy(out_hbm.at[cid, pl.ds(row, RB)], obuf)

            @pl.loop(0, RB)
            def _(rr):
                dv = plsc.load_gather(dinv_own, [_i16(b * RB + rr)])
                for f in range(FG):
                    w = wbuf[rr, pl.ds(f * 16, 16)]
                    obuf[rr, pl.ds(f * 16, 16)] = obuf[rr, pl.ds(f * 16, 16)] + gj * w
                    if j < K:
                        wbuf[rr, pl.ds(f * 16, 16)] = w * dv
            pltpu.sync_copy(obuf, out_hbm.at[cid, pl.ds(row, RB)])
            if j < K:
                pltpu.sync_copy(wbuf, nxt.at[pl.ds(row, RB)])
                pltpu.sync_copy(zbuf, cur.at[pl.ds(row, RB)])

        plsc.subcore_barrier()
        bufs = (bufs[1], bufs[0])

    g0 = gv[0, pl.ds(0, 16)]

    @pl.loop(0, nblk)
    def _(b, g0=g0):
        row = r0 + b * RB
        pltpu.sync_copy(h_hbm.at[cid, pl.ds(row, RB)], hbuf)
        pltpu.sync_copy(out_hbm.at[cid, pl.ds(row, RB)], obuf)

        @pl.loop(0, RB)
        def _(rr):
            dv = plsc.load_gather(dis_own, [_i16(b * RB + rr)])
            for f in range(FG):
                obuf[rr, pl.ds(f * 16, 16)] = (
                    g0 * hbuf[rr, pl.ds(f * 16, 16)]
                    + dv * obuf[rr, pl.ds(f * 16, 16)]
                )
        pltpu.sync_copy(obuf, out_hbm.at[cid, pl.ds(row, RB)])


def _make_sc_bern():
    return pl.kernel(
        _sc_body,
        out_type=jax.ShapeDtypeStruct((NC, N, DH), jnp.float32),
        mesh=plsc.VectorSubcoreMesh(core_axis_name="c", subcore_axis_name="s"),
        compiler_params=pltpu.CompilerParams(
            use_tc_tiling_on_sc=False, needs_layout_passes=False
        ),
        scratch_types=[
            pltpu.VMEM_SHARED((N, DH), jnp.float32),
            pltpu.VMEM_SHARED((N, DH), jnp.float32),
            pltpu.VMEM_SHARED((NT, NPAD), jnp.float32),
            pltpu.VMEM((N,), jnp.float32),
            pltpu.VMEM((2, CHUNK), jnp.int32),
            pltpu.VMEM((2, CHUNK), jnp.int32),
            pltpu.VMEM((CHUNK, DH), jnp.float32),
            pltpu.VMEM((CHUNK, DH), jnp.float32),
            pltpu.VMEM((RB, DH), jnp.float32),
            pltpu.VMEM((RB, DH), jnp.float32),
            pltpu.VMEM((RB, DH), jnp.float32),
            pltpu.VMEM((RB, DH), jnp.float32),
            pltpu.VMEM((RPT,), jnp.float32),
            pltpu.VMEM((RPT,), jnp.float32),
            pltpu.VMEM((RPT,), jnp.float32),
            pltpu.VMEM((RPT,), jnp.float32),
            pltpu.VMEM((16, 16), jnp.float32),
            pltpu.SemaphoreType.DMA,
            pltpu.SemaphoreType.DMA,
        ],
    )


def kernel(x, edge_index, W1, b1, W2, b2, temp):
    h = pl.pallas_call(
        _mlp_body,
        grid=(10,),
        in_specs=[
            pl.BlockSpec((1000, DF), lambda i: (i, 0)),
            pl.BlockSpec((DF, DO), lambda i: (0, 0)),
            pl.BlockSpec((1, DO), lambda i: (0, 0)),
            pl.BlockSpec((DO, DO), lambda i: (0, 0)),
            pl.BlockSpec((1, DO), lambda i: (0, 0)),
        ],
        out_specs=pl.BlockSpec((1000, DO), lambda i: (i, 0)),
        out_shape=jax.ShapeDtypeStruct((N, DO), jnp.float32),
    )(x, W1, b1[None, :], W2, b2[None, :])

    tr = jax.nn.relu(temp)
    g = jnp.sum(jnp.asarray(_GMAT) * tr[None, :], axis=1)
    g16 = jnp.zeros((16, 16), jnp.float32).at[: K + 1, :].set(
        jnp.broadcast_to(g[:, None], (K + 1, 16))
    )

    h2 = h.reshape(N, NC, DH).transpose(1, 0, 2)
    ei2 = jnp.pad(
        edge_index.reshape(2, NCH, CHUNK).transpose(1, 0, 2),
        ((0, NCHPAD - NCH), (0, 0), (0, 0)),
    )

    out2 = _make_sc_bern()(h2, ei2, g16)
    out_lin = out2.transpose(1, 0, 2).reshape(N, DO)

    return pl.pallas_call(
        _lsm_body,
        grid=(10,),
        in_specs=[pl.BlockSpec((1000, DO), lambda i: (i, 0))],
        out_specs=pl.BlockSpec((1000, DO), lambda i: (i, 0)),
        out_shape=jax.ShapeDtypeStruct((N, DO), jnp.float32),
    )(out_lin)


# speedup vs baseline: 63.3847x; 1.0318x over previous
from math import comb

import numpy as np
import jax
import jax.numpy as jnp
from jax import lax
from jax.experimental import pallas as pl
from jax.experimental.pallas import tpu as pltpu
from jax.experimental.pallas import tpu_sc as plsc

N = 10000
E = 320000
DF = 128
DO = 64
K = 10

NC = 2
DH = DO // NC
FG = DH // 16
NT = 16
RPT = 640
RB = 80
CHUNK = 128
NCH = E // CHUNK
CPT = NCH // NT
XTRA = NCH % NT
NCHPAD = NCH + 8
NPAD = 10240


def _coef_matrix() -> np.ndarray:
    G = np.zeros((K + 1, K + 1), np.float64)
    for k in range(K + 1):
        ck = comb(K, k) / 2.0**K
        for j in range(K + 1):
            s = 0
            for m in range(0, min(j, k) + 1):
                if j - m <= K - k:
                    s += (-1) ** m * comb(k, m) * comb(K - k, j - m)
            G[j, k] = s * ck
    return G.astype(np.float32)


_GMAT = _coef_matrix()


def _mlp_body(x_ref, w1_ref, b1_ref, w2_ref, b2_ref, o_ref):
    a = jnp.dot(x_ref[...], w1_ref[...], preferred_element_type=jnp.float32)
    a = jnp.maximum(a + b1_ref[...], 0.0)
    o_ref[...] = (
        jnp.dot(a, w2_ref[...], preferred_element_type=jnp.float32) + b2_ref[...]
    )


def _lsm_body(o_ref, y_ref):
    v = o_ref[...]
    m = jnp.max(v, axis=1, keepdims=True)
    e = jnp.exp(v - m)
    s = jnp.sum(e, axis=1, keepdims=True)
    y_ref[...] = v - m - jnp.log(s)


def _i16(v):
    return jnp.zeros((16,), jnp.int32) + v


def _rsqrt16(d):
    i = plsc.bitcast(d, jnp.int32)
    i = jnp.int32(0x5F3759DF) - lax.shift_right_arithmetic(i, 1)
    y = plsc.bitcast(i, jnp.float32)
    for _ in range(3):
        y = y * (1.5 - 0.5 * d * y * y)
    return y


def _sc_body(h_hbm, ei_hbm, g_hbm, out_hbm,
             A0, A1, OACC, hstage,
             hist, idx0, idx1, rowbuf0, rowbuf1,
             wbuf, hbuf, obuf, zbuf, accd, tbuf, dis_own, dinv_own,
             gv, sem0, sem1):
    cid = lax.axis_index("c")
    t = lax.axis_index("s")
    r0 = t * RPT
    nblk = jnp.minimum(RPT, N - r0) // RB
    cbase = t * CPT + jnp.minimum(t, XTRA)
    has_extra = t < XTRA
    Z16 = jnp.zeros((16,), jnp.float32)
    ONES16 = jnp.ones((16,), jnp.float32)

    def load_idx(sd_ref, c):
        pltpu.sync_copy(ei_hbm.at[c], sd_ref)

    pltpu.sync_copy(g_hbm, gv)

    @pl.loop(0, N // 16)
    def _(i):
        hist[pl.ds(i * 16, 16)] = Z16

    @pl.loop(0, (RB * DH) // 16)
    def _(i):
        zbuf[i // FG, pl.ds((i % FG) * 16, 16)] = Z16

    @pl.loop(0, CPT + has_extra.astype(jnp.int32))
    def _(ci):
        pltpu.sync_copy(ei_hbm.at[cbase + ci], idx0)

        @pl.loop(0, CHUNK // 16)
        def _(kk):
            idx = idx0[0, pl.ds(kk * 16, 16)]
            plsc.addupdate_scatter(hist, [idx], ONES16)

    pltpu.sync_copy(hist, hstage.at[t, pl.ds(0, N)])

    plsc.subcore_barrier()

    pltpu.sync_copy(hstage.at[0, pl.ds(r0, RPT)], accd)
    for tt in range(1, NT):
        pltpu.sync_copy(hstage.at[tt, pl.ds(r0, RPT)], tbuf)

        @pl.loop(0, RPT // 16)
        def _(i):
            accd[pl.ds(i * 16, 16)] = accd[pl.ds(i * 16, 16)] + tbuf[pl.ds(i * 16, 16)]

    @pl.loop(0, RPT // 16)
    def _(i):
        d = accd[pl.ds(i * 16, 16)]
        m = d > 0.0
        y = _rsqrt16(jnp.where(m, d, 1.0))
        dis = jnp.where(m, y, 0.0)
        dis_own[pl.ds(i * 16, 16)] = dis
        dinv_own[pl.ds(i * 16, 16)] = dis * dis

    @pl.loop(0, nblk)
    def _(b):
        row = r0 + b * RB
        pltpu.sync_copy(h_hbm.at[cid, pl.ds(row, RB)], hbuf)

        @pl.loop(0, RB)
        def _(rr):
            dv = plsc.load_gather(dis_own, [_i16(b * RB + rr)])
            for f in range(FG):
                hbuf[rr, pl.ds(f * 16, 16)] = hbuf[rr, pl.ds(f * 16, 16)] * dv
        pltpu.sync_copy(hbuf, A0.at[pl.ds(row, RB)])
        pltpu.sync_copy(zbuf, A1.at[pl.ds(row, RB)])
        pltpu.sync_copy(zbuf, OACC.at[pl.ds(row, RB)])

    plsc.subcore_barrier()

    bufs = (A0, A1)
    for j in range(1, K + 1):
        cur, nxt = bufs

        load_idx(idx0, cbase)
        pltpu.async_copy(cur.at[idx0.at[0]], rowbuf0, sem0)

        @pl.loop(0, CPT // 2)
        def _(i, cur=cur, nxt=nxt):
            c = cbase + 2 * i
            load_idx(idx1, c + 1)
            pltpu.make_async_copy(cur.at[idx0.at[0]], rowbuf0, sem0).wait()
            pltpu.async_copy(cur.at[idx1.at[0]], rowbuf1, sem1)
            pltpu.sync_copy(rowbuf0, nxt.at[idx0.at[1]], add=True)
            load_idx(idx0, c + 2)
            pltpu.make_async_copy(cur.at[idx1.at[0]], rowbuf1, sem1).wait()
            pltpu.async_copy(cur.at[idx0.at[0]], rowbuf0, sem0)
            pltpu.sync_copy(rowbuf1, nxt.at[idx1.at[1]], add=True)

        pltpu.make_async_copy(cur.at[idx0.at[0]], rowbuf0, sem0).wait()

        @pl.when(has_extra)
        def _(cur=cur, nxt=nxt):
            pltpu.sync_copy(rowbuf0, nxt.at[idx0.at[1]], add=True)

        plsc.subcore_barrier()

        gj = gv[j, pl.ds(0, 16)]

        @pl.loop(0, nblk)
        def _(b, cur=cur, nxt=nxt, j=j, gj=gj):
            row = r0 + b * RB
            pltpu.sync_copy(nxt.at[pl.ds(row, RB)], wbuf)
            pltpu.sync_copy(OACC.at[pl.ds(row, RB)], obuf)

            @pl.loop(0, RB)
            def _(rr):
                dv = plsc.load_gather(dinv_own, [_i16(b * RB + rr)])
                for f in range(FG):
                    w = wbuf[rr, pl.ds(f * 16, 16)]
                    obuf[rr, pl.ds(f * 16, 16)] = obuf[rr, pl.ds(f * 16, 16)] + gj * w
                    if j < K:
                        wbuf[rr, pl.ds(f * 16, 16)] = w * dv
            pltpu.sync_copy(obuf, OACC.at[pl.ds(row, RB)])
            if j < K:
                pltpu.sync_copy(wbuf, nxt.at[pl.ds(row, RB)])
                pltpu.sync_copy(zbuf, cur.at[pl.ds(row, RB)])

        plsc.subcore_barrier()
        bufs = (bufs[1], bufs[0])

    g0 = gv[0, pl.ds(0, 16)]

    @pl.loop(0, nblk)
    def _(b, g0=g0):
        row = r0 + b * RB
        pltpu.sync_copy(h_hbm.at[cid, pl.ds(row, RB)], hbuf)
        pltpu.sync_copy(OACC.at[pl.ds(row, RB)], obuf)

        @pl.loop(0, RB)
        def _(rr):
            dv = plsc.load_gather(dis_own, [_i16(b * RB + rr)])
            for f in range(FG):
                obuf[rr, pl.ds(f * 16, 16)] = (
                    g0 * hbuf[rr, pl.ds(f * 16, 16)]
                    + dv * obuf[rr, pl.ds(f * 16, 16)]
                )
        pltpu.sync_copy(obuf, out_hbm.at[cid, pl.ds(row, RB)])


def _make_sc_bern():
    return pl.kernel(
        _sc_body,
        out_type=jax.ShapeDtypeStruct((NC, N, DH), jnp.float32),
        mesh=plsc.VectorSubcoreMesh(core_axis_name="c", subcore_axis_name="s"),
        compiler_params=pltpu.CompilerParams(
            use_tc_tiling_on_sc=False, needs_layout_passes=False
        ),
        scratch_types=[
            pltpu.VMEM_SHARED((N, DH), jnp.float32),
            pltpu.VMEM_SHARED((N, DH), jnp.float32),
            pltpu.VMEM_SHARED((N, DH), jnp.float32),
            pltpu.VMEM_SHARED((NT, NPAD), jnp.float32),
            pltpu.VMEM((N,), jnp.float32),
            pltpu.VMEM((2, CHUNK), jnp.int32),
            pltpu.VMEM((2, CHUNK), jnp.int32),
            pltpu.VMEM((CHUNK, DH), jnp.float32),
            pltpu.VMEM((CHUNK, DH), jnp.float32),
            pltpu.VMEM((RB, DH), jnp.float32),
            pltpu.VMEM((RB, DH), jnp.float32),
            pltpu.VMEM((RB, DH), jnp.float32),
            pltpu.VMEM((RB, DH), jnp.float32),
            pltpu.VMEM((RPT,), jnp.float32),
            pltpu.VMEM((RPT,), jnp.float32),
            pltpu.VMEM((RPT,), jnp.float32),
            pltpu.VMEM((RPT,), jnp.float32),
            pltpu.VMEM((16, 16), jnp.float32),
            pltpu.SemaphoreType.DMA,
            pltpu.SemaphoreType.DMA,
        ],
    )


def kernel(x, edge_index, W1, b1, W2, b2, temp):
    h = pl.pallas_call(
        _mlp_body,
        grid=(10,),
        in_specs=[
            pl.BlockSpec((1000, DF), lambda i: (i, 0)),
            pl.BlockSpec((DF, DO), lambda i: (0, 0)),
            pl.BlockSpec((1, DO), lambda i: (0, 0)),
            pl.BlockSpec((DO, DO), lambda i: (0, 0)),
            pl.BlockSpec((1, DO), lambda i: (0, 0)),
        ],
        out_specs=pl.BlockSpec((1000, DO), lambda i: (i, 0)),
        out_shape=jax.ShapeDtypeStruct((N, DO), jnp.float32),
    )(x, W1, b1[None, :], W2, b2[None, :])

    tr = jax.nn.relu(temp)
    g = jnp.sum(jnp.asarray(_GMAT) * tr[None, :], axis=1)
    g16 = jnp.zeros((16, 16), jnp.float32).at[: K + 1, :].set(
        jnp.broadcast_to(g[:, None], (K + 1, 16))
    )

    h2 = h.reshape(N, NC, DH).transpose(1, 0, 2)
    ei2 = jnp.pad(
        edge_index.reshape(2, NCH, CHUNK).transpose(1, 0, 2),
        ((0, NCHPAD - NCH), (0, 0), (0, 0)),
    )

    out2 = _make_sc_bern()(h2, ei2, g16)
    out_lin = out2.transpose(1, 0, 2).reshape(N, DO)

    return pl.pallas_call(
        _lsm_body,
        grid=(10,),
        in_specs=[pl.BlockSpec((1000, DO), lambda i: (i, 0))],
        out_specs=pl.BlockSpec((1000, DO), lambda i: (i, 0)),
        out_shape=jax.ShapeDtypeStruct((N, DO), jnp.float32),
    )(out_lin)


# speedup vs baseline: 65.1512x; 1.0279x over previous
from math import comb

import numpy as np
import jax
import jax.numpy as jnp
from jax import lax
from jax.experimental import pallas as pl
from jax.experimental.pallas import tpu as pltpu
from jax.experimental.pallas import tpu_sc as plsc

N = 10000
E = 320000
DF = 128
DO = 64
K = 10

NC = 2
DH = DO // NC
FG = DH // 16
NT = 16
RPT = 640
RB = 80
CHUNK = 128
NCH = E // CHUNK
CPT = NCH // NT
XTRA = NCH % NT
NCHPAD = NCH + 8
NPAD = 10240


def _coef_matrix() -> np.ndarray:
    G = np.zeros((K + 1, K + 1), np.float64)
    for k in range(K + 1):
        ck = comb(K, k) / 2.0**K
        for j in range(K + 1):
            s = 0
            for m in range(0, min(j, k) + 1):
                if j - m <= K - k:
                    s += (-1) ** m * comb(k, m) * comb(K - k, j - m)
            G[j, k] = s * ck
    return G.astype(np.float32)


_GMAT = _coef_matrix()


def _mlp_body(x_ref, w1_ref, b1_ref, w2_ref, b2_ref, o_ref):
    a = jnp.dot(x_ref[...], w1_ref[...], preferred_element_type=jnp.float32)
    a = jnp.maximum(a + b1_ref[...], 0.0)
    o_ref[...] = (
        jnp.dot(a, w2_ref[...], preferred_element_type=jnp.float32) + b2_ref[...]
    )


def _lsm_body(o_ref, y_ref):
    v = o_ref[...]
    m = jnp.max(v, axis=1, keepdims=True)
    e = jnp.exp(v - m)
    s = jnp.sum(e, axis=1, keepdims=True)
    y_ref[...] = v - m - jnp.log(s)


def _i16(v):
    return jnp.zeros((16,), jnp.int32) + v


def _rsqrt16(d):
    i = plsc.bitcast(d, jnp.int32)
    i = jnp.int32(0x5F3759DF) - lax.shift_right_arithmetic(i, 1)
    y = plsc.bitcast(i, jnp.float32)
    for _ in range(3):
        y = y * (1.5 - 0.5 * d * y * y)
    return y


def _sc_body(h_hbm, ei_hbm, g_hbm, out_hbm,
             A0, A1, OACC, hstage,
             hist, idx0, idx1, rowbuf0, rowbuf1,
             wbuf, hbuf, obuf, zbuf, accd, tbuf, dis_own, dinv_own,
             gv, sem0, sem1):
    cid = lax.axis_index("c")
    t = lax.axis_index("s")
    r0 = t * RPT
    nblk = jnp.minimum(RPT, N - r0) // RB
    cbase = t * CPT + jnp.minimum(t, XTRA)
    has_extra = t < XTRA
    Z16 = jnp.zeros((16,), jnp.float32)
    ONES16 = jnp.ones((16,), jnp.float32)

    def load_idx(sd_ref, c):
        pltpu.sync_copy(ei_hbm.at[c], sd_ref)

    pltpu.sync_copy(g_hbm, gv)

    @pl.loop(0, N // 16)
    def _(i):
        hist[pl.ds(i * 16, 16)] = Z16

    @pl.loop(0, (RB * DH) // 16)
    def _(i):
        zbuf[i // FG, pl.ds((i % FG) * 16, 16)] = Z16

    def hist_chunk(sd_ref):
        @pl.loop(0, CHUNK // 16)
        def _(kk):
            idx = sd_ref[0, pl.ds(kk * 16, 16)]
            plsc.addupdate_scatter(hist, [idx], ONES16)

    pltpu.async_copy(ei_hbm.at[cbase], idx0, sem0)

    @pl.loop(0, CPT // 2)
    def _(i):
        c = cbase + 2 * i
        pltpu.async_copy(ei_hbm.at[c + 1], idx1, sem1)
        pltpu.make_async_copy(ei_hbm.at[c], idx0, sem0).wait()
        hist_chunk(idx0)
        pltpu.async_copy(ei_hbm.at[c + 2], idx0, sem0)
        pltpu.make_async_copy(ei_hbm.at[c + 1], idx1, sem1).wait()
        hist_chunk(idx1)

    pltpu.make_async_copy(ei_hbm.at[cbase], idx0, sem0).wait()

    @pl.when(has_extra)
    def _():
        hist_chunk(idx0)

    pltpu.sync_copy(hist, hstage.at[t, pl.ds(0, N)])

    plsc.subcore_barrier()

    pltpu.sync_copy(hstage.at[0, pl.ds(r0, RPT)], accd)
    for tt in range(1, NT):
        pltpu.sync_copy(hstage.at[tt, pl.ds(r0, RPT)], tbuf)

        @pl.loop(0, RPT // 16)
        def _(i):
            accd[pl.ds(i * 16, 16)] = accd[pl.ds(i * 16, 16)] + tbuf[pl.ds(i * 16, 16)]

    @pl.loop(0, RPT // 16)
    def _(i):
        d = accd[pl.ds(i * 16, 16)]
        m = d > 0.0
        y = _rsqrt16(jnp.where(m, d, 1.0))
        dis = jnp.where(m, y, 0.0)
        dis_own[pl.ds(i * 16, 16)] = dis
        dinv_own[pl.ds(i * 16, 16)] = dis * dis

    @pl.loop(0, nblk)
    def _(b):
        row = r0 + b * RB
        pltpu.sync_copy(h_hbm.at[cid, pl.ds(row, RB)], hbuf)

        @pl.loop(0, RB)
        def _(rr):
            dv = plsc.load_gather(dis_own, [_i16(b * RB + rr)])
            for f in range(FG):
                hbuf[rr, pl.ds(f * 16, 16)] = hbuf[rr, pl.ds(f * 16, 16)] * dv
        pltpu.sync_copy(hbuf, A0.at[pl.ds(row, RB)])
        pltpu.sync_copy(zbuf, A1.at[pl.ds(row, RB)])
        pltpu.sync_copy(zbuf, OACC.at[pl.ds(row, RB)])

    plsc.subcore_barrier()

    bufs = (A0, A1)
    for j in range(1, K + 1):
        cur, nxt = bufs

        load_idx(idx0, cbase)
        pltpu.async_copy(cur.at[idx0.at[0]], rowbuf0, sem0)

        @pl.loop(0, CPT // 2)
        def _(i, cur=cur, nxt=nxt):
            c = cbase + 2 * i
            load_idx(idx1, c + 1)
            pltpu.make_async_copy(cur.at[idx0.at[0]], rowbuf0, sem0).wait()
            pltpu.async_copy(cur.at[idx1.at[0]], rowbuf1, sem1)
            pltpu.sync_copy(rowbuf0, nxt.at[idx0.at[1]], add=True)
            load_idx(idx0, c + 2)
            pltpu.make_async_copy(cur.at[idx1.at[0]], rowbuf1, sem1).wait()
            pltpu.async_copy(cur.at[idx0.at[0]], rowbuf0, sem0)
            pltpu.sync_copy(rowbuf1, nxt.at[idx1.at[1]], add=True)

        pltpu.make_async_copy(cur.at[idx0.at[0]], rowbuf0, sem0).wait()

        @pl.when(has_extra)
        def _(cur=cur, nxt=nxt):
            pltpu.sync_copy(rowbuf0, nxt.at[idx0.at[1]], add=True)

        plsc.subcore_barrier()

        gj = gv[j, pl.ds(0, 16)]

        @pl.loop(0, nblk)
        def _(b, cur=cur, nxt=nxt, j=j, gj=gj):
            row = r0 + b * RB
            pltpu.sync_copy(nxt.at[pl.ds(row, RB)], wbuf)
            pltpu.sync_copy(OACC.at[pl.ds(row, RB)], obuf)

            @pl.loop(0, RB)
            def _(rr):
                dv = plsc.load_gather(dinv_own, [_i16(b * RB + rr)])
                for f in range(FG):
                    w = wbuf[rr, pl.ds(f * 16, 16)]
                    obuf[rr, pl.ds(f * 16, 16)] = obuf[rr, pl.ds(f * 16, 16)] + gj * w
                    if j < K:
                        wbuf[rr, pl.ds(f * 16, 16)] = w * dv
            pltpu.sync_copy(obuf, OACC.at[pl.ds(row, RB)])
            if j < K:
                pltpu.sync_copy(wbuf, nxt.at[pl.ds(row, RB)])
                pltpu.sync_copy(zbuf, cur.at[pl.ds(row, RB)])

        plsc.subcore_barrier()
        bufs = (bufs[1], bufs[0])

    g0 = gv[0, pl.ds(0, 16)]

    @pl.loop(0, nblk)
    def _(b, g0=g0):
        row = r0 + b * RB
        pltpu.sync_copy(h_hbm.at[cid, pl.ds(row, RB)], hbuf)
        pltpu.sync_copy(OACC.at[pl.ds(row, RB)], obuf)

        @pl.loop(0, RB)
        def _(rr):
            dv = plsc.load_gather(dis_own, [_i16(b * RB + rr)])
            for f in range(FG):
                obuf[rr, pl.ds(f * 16, 16)] = (
                    g0 * hbuf[rr, pl.ds(f * 16, 16)]
                    + dv * obuf[rr, pl.ds(f * 16, 16)]
                )
        pltpu.sync_copy(obuf, out_hbm.at[cid, pl.ds(row, RB)])


def _make_sc_bern():
    return pl.kernel(
        _sc_body,
        out_type=jax.ShapeDtypeStruct((NC, N, DH), jnp.float32),
        mesh=plsc.VectorSubcoreMesh(core_axis_name="c", subcore_axis_name="s"),
        compiler_params=pltpu.CompilerParams(
            use_tc_tiling_on_sc=False, needs_layout_passes=False
        ),
        scratch_types=[
            pltpu.VMEM_SHARED((N, DH), jnp.float32),
            pltpu.VMEM_SHARED((N, DH), jnp.float32),
            pltpu.VMEM_SHARED((N, DH), jnp.float32),
            pltpu.VMEM_SHARED((NT, NPAD), jnp.float32),
            pltpu.VMEM((N,), jnp.float32),
            pltpu.VMEM((2, CHUNK), jnp.int32),
            pltpu.VMEM((2, CHUNK), jnp.int32),
            pltpu.VMEM((CHUNK, DH), jnp.float32),
            pltpu.VMEM((CHUNK, DH), jnp.float32),
            pltpu.VMEM((RB, DH), jnp.float32),
            pltpu.VMEM((RB, DH), jnp.float32),
            pltpu.VMEM((RB, DH), jnp.float32),
            pltpu.VMEM((RB, DH), jnp.float32),
            pltpu.VMEM((RPT,), jnp.float32),
            pltpu.VMEM((RPT,), jnp.float32),
            pltpu.VMEM((RPT,), jnp.float32),
            pltpu.VMEM((RPT,), jnp.float32),
            pltpu.VMEM((16, 16), jnp.float32),
            pltpu.SemaphoreType.DMA,
            pltpu.SemaphoreType.DMA,
        ],
    )


def kernel(x, edge_index, W1, b1, W2, b2, temp):
    h = pl.pallas_call(
        _mlp_body,
        grid=(10,),
        in_specs=[
            pl.BlockSpec((1000, DF), lambda i: (i, 0)),
            pl.BlockSpec((DF, DO), lambda i: (0, 0)),
            pl.BlockSpec((1, DO), lambda i: (0, 0)),
            pl.BlockSpec((DO, DO), lambda i: (0, 0)),
            pl.BlockSpec((1, DO), lambda i: (0, 0)),
        ],
        out_specs=pl.BlockSpec((1000, DO), lambda i: (i, 0)),
        out_shape=jax.ShapeDtypeStruct((N, DO), jnp.float32),
    )(x, W1, b1[None, :], W2, b2[None, :])

    tr = jax.nn.relu(temp)
    g = jnp.sum(jnp.asarray(_GMAT) * tr[None, :], axis=1)
    g16 = jnp.zeros((16, 16), jnp.float32).at[: K + 1, :].set(
        jnp.broadcast_to(g[:, None], (K + 1, 16))
    )

    h2 = h.reshape(N, NC, DH).transpose(1, 0, 2)
    ei2 = jnp.pad(
        edge_index.reshape(2, NCH, CHUNK).transpose(1, 0, 2),
        ((0, NCHPAD - NCH), (0, 0), (0, 0)),
    )

    out2 = _make_sc_bern()(h2, ei2, g16)
    out_lin = out2.transpose(1, 0, 2).reshape(N, DO)

    return pl.pallas_call(
        _lsm_body,
        grid=(10,),
        in_specs=[pl.BlockSpec((1000, DO), lambda i: (i, 0))],
        out_specs=pl.BlockSpec((1000, DO), lambda i: (i, 0)),
        out_shape=jax.ShapeDtypeStruct((N, DO), jnp.float32),
    )(out_lin)
